# Initial kernel scaffold; baseline (speedup 1.0000x reference)
import jax
import jax.numpy as jnp
from jax.experimental import pallas as pl


def kernel(x1, x2, sldj_x, fwd_idxs):
    raise NotImplementedError("write your pallas kernel here")


# speedup vs baseline: 2.0231x; 2.0231x over previous
import jax
import jax.numpy as jnp
from jax.experimental import pallas as pl
from jax.experimental.pallas import tpu as pltpu

B, C_HALF, H, W = 32, 192, 64, 64
C_TOTAL = 2 * C_HALF
SUB, LANE = 32, 128


def _shuffle_body(o1_map, o2_map, dsel, x1_ref, x2_ref, out1_ref, out2_ref):
    i = pl.program_id(0)
    from_x2 = i >= C_HALF
    to_out2 = dsel[i] == 1

    @pl.when(jnp.logical_and(jnp.logical_not(from_x2), jnp.logical_not(to_out2)))
    def _():
        out1_ref[...] = x1_ref[...]

    @pl.when(jnp.logical_and(jnp.logical_not(from_x2), to_out2))
    def _():
        out2_ref[...] = x1_ref[...]

    @pl.when(jnp.logical_and(from_x2, jnp.logical_not(to_out2)))
    def _():
        out1_ref[...] = x2_ref[...]

    @pl.when(jnp.logical_and(from_x2, to_out2))
    def _():
        out2_ref[...] = x2_ref[...]


def _fill_forward(dst, mask):
    steps = jnp.arange(C_TOTAL, dtype=jnp.int32)
    marked = jnp.where(mask, steps, -1)
    last = jax.lax.cummax(marked)
    first = jnp.argmax(mask).astype(jnp.int32)
    idx = jnp.where(last >= 0, last, first)
    return dst[idx]


def kernel(x1, x2, sldj_x, fwd_idxs):
    x1r = x1.reshape(B, C_HALF, SUB, LANE)
    x2r = x2.reshape(B, C_HALF, SUB, LANE)

    dst = jnp.argsort(fwd_idxs).astype(jnp.int32)
    in_out1 = dst < C_HALF
    o1_map = _fill_forward(dst, in_out1)
    o2_map = _fill_forward(dst - C_HALF, jnp.logical_not(in_out1))
    dsel = jnp.logical_not(in_out1).astype(jnp.int32)

    block = (B, 1, SUB, LANE)
    grid_spec = pltpu.PrefetchScalarGridSpec(
        num_scalar_prefetch=3,
        grid=(C_TOTAL,),
        in_specs=[
            pl.BlockSpec(block, lambda i, o1, o2, ds: (0, jnp.minimum(i, C_HALF - 1), 0, 0)),
            pl.BlockSpec(block, lambda i, o1, o2, ds: (0, jnp.maximum(i - C_HALF, 0), 0, 0)),
        ],
        out_specs=[
            pl.BlockSpec(block, lambda i, o1, o2, ds: (0, o1[i], 0, 0)),
            pl.BlockSpec(block, lambda i, o1, o2, ds: (0, o2[i], 0, 0)),
        ],
    )

    out_shape = jax.ShapeDtypeStruct((B, C_HALF, SUB, LANE), jnp.float32)
    out1, out2 = pl.pallas_call(
        _shuffle_body,
        grid_spec=grid_spec,
        out_shape=[out_shape, out_shape],
    )(o1_map, o2_map, dsel, x1r, x2r)

    return (
        out1.reshape(B, C_HALF, H, W),
        out2.reshape(B, C_HALF, H, W),
        sldj_x,
    )
